# Initial kernel scaffold; baseline (speedup 1.0000x reference)
import jax
import jax.numpy as jnp
from jax.experimental import pallas as pl
from jax.experimental.pallas import tpu as pltpu

B = 2048
D = 768
H = 768
OUT = 768
E = 8
G4 = 4 * H
TCH = 256
NT = B // TCH

NEG_INF = -1e30


def _scan_body(x_ref, wih_ref, whh_ref, bsum_ref, hs_ref, xg_scr, h_scr, c_scr):
    t = pl.program_id(1)

    xg_scr[...] = jax.lax.dot_general(
        x_ref[...], wih_ref[0],
        dimension_numbers=(((1,), (1,)), ((), ())),
        preferred_element_type=jnp.float32,
    ) + bsum_ref[0]

    @pl.when(t == 0)
    def _():
        h_scr[...] = jnp.zeros_like(h_scr)
        c_scr[...] = jnp.zeros_like(c_scr)

    whh = whh_ref[0]

    def step(i, carry):
        h, c = carry
        gates = xg_scr[pl.ds(i, 1), :] + jax.lax.dot_general(
            h, whh,
            dimension_numbers=(((1,), (1,)), ((), ())),
            preferred_element_type=jnp.float32,
        )
        ig = jax.nn.sigmoid(gates[:, 0:H])
        fg = jax.nn.sigmoid(gates[:, H:2 * H])
        gg = jnp.tanh(gates[:, 2 * H:3 * H])
        og = jax.nn.sigmoid(gates[:, 3 * H:4 * H])
        c2 = fg * c + ig * gg
        h2 = og * jnp.tanh(c2)
        hs_ref[pl.ds(i, 1), :] = h2
        return h2, c2

    h0 = h_scr[...]
    c0 = c_scr[...]
    h_fin, c_fin = jax.lax.fori_loop(0, TCH, step, (h0, c0))
    h_scr[...] = h_fin
    c_scr[...] = c_fin


def _combine_body(x_ref, wg_ref, bg_ref, bout_ref, hs_ref, wout_ref, out_ref,
                  gmask_scr):
    e = pl.program_id(0)

    @pl.when(e == 0)
    def _():
        logits = jax.lax.dot_general(
            x_ref[...], wg_ref[...],
            dimension_numbers=(((1,), (1,)), ((), ())),
            preferred_element_type=jnp.float32,
        ) + bg_ref[0]
        m = jnp.max(logits, axis=1, keepdims=True)
        ex = jnp.exp(logits - m)
        gating = ex / jnp.sum(ex, axis=1, keepdims=True)
        idx = jax.lax.broadcasted_iota(jnp.int32, (B, E), 1)
        a1 = jnp.argmax(gating, axis=1)[:, None]
        sel1 = idx == a1
        g2 = jnp.where(sel1, NEG_INF, gating)
        a2 = jnp.argmax(g2, axis=1)[:, None]
        mask = jnp.logical_or(sel1, idx == a2)
        gmask_scr[...] = jnp.where(mask, gating, 0.0)
        out_ref[...] = jnp.zeros_like(out_ref)

    gcol = jax.lax.dynamic_slice(gmask_scr[...], (0, e), (B, 1))
    weighted_h = hs_ref[0] * gcol
    out_ref[...] += jax.lax.dot_general(
        weighted_h, wout_ref[0],
        dimension_numbers=(((1,), (1,)), ((), ())),
        preferred_element_type=jnp.float32,
    )

    @pl.when(e == E - 1)
    def _():
        out_ref[...] += jnp.dot(gmask_scr[...], bout_ref[...],
                                preferred_element_type=jnp.float32)


def kernel(x, Wg, bg, W_ih, W_hh, b_ih, b_hh, W_out, b_out):
    bsum = (b_ih + b_hh).reshape(E, 1, G4)

    hs = pl.pallas_call(
        _scan_body,
        grid=(E, NT),
        in_specs=[
            pl.BlockSpec((TCH, D), lambda e, t: (t, 0)),
            pl.BlockSpec((1, G4, D), lambda e, t: (e, 0, 0)),
            pl.BlockSpec((1, G4, H), lambda e, t: (e, 0, 0)),
            pl.BlockSpec((1, 1, G4), lambda e, t: (e, 0, 0)),
        ],
        out_specs=pl.BlockSpec((1, TCH, H), lambda e, t: (e, t, 0)),
        out_shape=jax.ShapeDtypeStruct((E, B, H), jnp.float32),
        scratch_shapes=[
            pltpu.VMEM((TCH, G4), jnp.float32),
            pltpu.VMEM((1, H), jnp.float32),
            pltpu.VMEM((1, H), jnp.float32),
        ],
    )(x, W_ih, W_hh, bsum)

    out = pl.pallas_call(
        _combine_body,
        grid=(E,),
        in_specs=[
            pl.BlockSpec((B, D), lambda e: (0, 0)),
            pl.BlockSpec((E, D), lambda e: (0, 0)),
            pl.BlockSpec((1, E), lambda e: (0, 0)),
            pl.BlockSpec((E, OUT), lambda e: (0, 0)),
            pl.BlockSpec((1, B, H), lambda e: (e, 0, 0)),
            pl.BlockSpec((1, OUT, H), lambda e: (e, 0, 0)),
        ],
        out_specs=pl.BlockSpec((B, OUT), lambda e: (0, 0)),
        out_shape=jax.ShapeDtypeStruct((B, OUT), jnp.float32),
        scratch_shapes=[
            pltpu.VMEM((B, E), jnp.float32),
        ],
    )(x, Wg, bg.reshape(1, E), b_out, hs, W_out)

    return out


# speedup vs baseline: 1.8108x; 1.8108x over previous
import jax
import jax.numpy as jnp
from jax.experimental import pallas as pl
from jax.experimental.pallas import tpu as pltpu

B = 2048
D = 768
H = 768
OUT = 768
E = 8
G4 = 4 * H
TCH = 256
NT = B // TCH

NEG_INF = -1e30


def _scan_body(x_ref, wih_ref, whh_ref, bsum_ref, hs_ref, xg_scr, h_scr, c_scr):
    t = pl.program_id(1)

    xg_scr[...] = jax.lax.dot_general(
        x_ref[...], wih_ref[0],
        dimension_numbers=(((1,), (1,)), ((), ())),
        preferred_element_type=jnp.float32,
    ) + bsum_ref[0]

    @pl.when(t == 0)
    def _():
        h_scr[...] = jnp.zeros_like(h_scr)
        c_scr[...] = jnp.zeros_like(c_scr)

    whh = whh_ref[0]

    def step(i, carry):
        h, c = carry
        gates = xg_scr[pl.ds(i, 1), :] + jax.lax.dot_general(
            h, whh,
            dimension_numbers=(((1,), (1,)), ((), ())),
            preferred_element_type=jnp.float32,
        )
        ig = jax.nn.sigmoid(gates[:, 0:H])
        fg = jax.nn.sigmoid(gates[:, H:2 * H])
        gg = jnp.tanh(gates[:, 2 * H:3 * H])
        og = jax.nn.sigmoid(gates[:, 3 * H:4 * H])
        c2 = fg * c + ig * gg
        h2 = og * jnp.tanh(c2)
        hs_ref[0, pl.ds(i, 1), :] = h2
        return h2, c2

    h0 = h_scr[...]
    c0 = c_scr[...]
    h_fin, c_fin = jax.lax.fori_loop(0, TCH, step, (h0, c0))
    h_scr[...] = h_fin
    c_scr[...] = c_fin


def _combine_body(x_ref, wg_ref, bg_ref, bout_ref, hs_ref, wout_ref, out_ref,
                  gmask_scr):
    e = pl.program_id(0)

    @pl.when(e == 0)
    def _():
        logits = jax.lax.dot_general(
            x_ref[...], wg_ref[...],
            dimension_numbers=(((1,), (1,)), ((), ())),
            preferred_element_type=jnp.float32,
        ) + bg_ref[0]
        m = jnp.max(logits, axis=1, keepdims=True)
        ex = jnp.exp(logits - m)
        gating = ex / jnp.sum(ex, axis=1, keepdims=True)
        idx = jax.lax.broadcasted_iota(jnp.int32, (B, E), 1)
        a1 = jnp.argmax(gating, axis=1)[:, None]
        sel1 = idx == a1
        g2 = jnp.where(sel1, NEG_INF, gating)
        a2 = jnp.argmax(g2, axis=1)[:, None]
        mask = jnp.logical_or(sel1, idx == a2)
        gmask_scr[...] = jnp.where(mask, gating, 0.0)
        out_ref[...] = jnp.zeros_like(out_ref)

    lane = jax.lax.broadcasted_iota(jnp.int32, (B, E), 1)
    gcol = jnp.sum(jnp.where(lane == e, gmask_scr[...], 0.0),
                   axis=1, keepdims=True)
    weighted_h = hs_ref[0] * gcol
    out_ref[...] += jax.lax.dot_general(
        weighted_h, wout_ref[0],
        dimension_numbers=(((1,), (1,)), ((), ())),
        preferred_element_type=jnp.float32,
    )

    @pl.when(e == E - 1)
    def _():
        out_ref[...] += jnp.dot(gmask_scr[...], bout_ref[...],
                                preferred_element_type=jnp.float32)


def kernel(x, Wg, bg, W_ih, W_hh, b_ih, b_hh, W_out, b_out):
    bsum = (b_ih + b_hh).reshape(E, 1, G4)

    hs = pl.pallas_call(
        _scan_body,
        grid=(E, NT),
        in_specs=[
            pl.BlockSpec((TCH, D), lambda e, t: (t, 0)),
            pl.BlockSpec((1, G4, D), lambda e, t: (e, 0, 0)),
            pl.BlockSpec((1, G4, H), lambda e, t: (e, 0, 0)),
            pl.BlockSpec((1, 1, G4), lambda e, t: (e, 0, 0)),
        ],
        out_specs=pl.BlockSpec((1, TCH, H), lambda e, t: (e, t, 0)),
        out_shape=jax.ShapeDtypeStruct((E, B, H), jnp.float32),
        scratch_shapes=[
            pltpu.VMEM((TCH, G4), jnp.float32),
            pltpu.VMEM((1, H), jnp.float32),
            pltpu.VMEM((1, H), jnp.float32),
        ],
    )(x, W_ih, W_hh, bsum)

    out = pl.pallas_call(
        _combine_body,
        grid=(E,),
        in_specs=[
            pl.BlockSpec((B, D), lambda e: (0, 0)),
            pl.BlockSpec((E, D), lambda e: (0, 0)),
            pl.BlockSpec((1, E), lambda e: (0, 0)),
            pl.BlockSpec((E, OUT), lambda e: (0, 0)),
            pl.BlockSpec((1, B, H), lambda e: (e, 0, 0)),
            pl.BlockSpec((1, OUT, H), lambda e: (e, 0, 0)),
        ],
        out_specs=pl.BlockSpec((B, OUT), lambda e: (0, 0)),
        out_shape=jax.ShapeDtypeStruct((B, OUT), jnp.float32),
        scratch_shapes=[
            pltpu.VMEM((B, E), jnp.float32),
        ],
    )(x, Wg, bg.reshape(1, E), b_out, hs, W_out)

    return out
